# baseline (speedup 1.0000x reference)
import functools

import jax
import jax.numpy as jnp
from jax import lax
from jax.experimental import pallas as pl
from jax.experimental.pallas import tpu as pltpu
from jax.experimental.pallas import tpu_sc as plsc

NN = 10000
EE = 160000
DIN = 256
DH = 512

NTILES = 16
NCORES = 2
BATCH = 128
RPT = 80
E_PAD = NTILES * BATCH * RPT
N_ACC = 10112
ROWS_PT = 632
ROWS_LAST = NN - ROWS_PT * (NTILES - 1)
ACC_PT = N_ACC // NTILES
DBATCH = 128
DEG_ROWS = E_PAD // DBATCH
NCH = 4
CW = DH // NCH
DEG_RPT = DEG_ROWS // (NCORES * NTILES)


def _split_rows(sid, fn):
    @pl.when(sid < NTILES - 1)
    def _():
        fn(pl.multiple_of(sid * ROWS_PT, 8), ROWS_PT)

    @pl.when(sid == NTILES - 1)
    def _():
        fn((NTILES - 1) * ROWS_PT, ROWS_LAST)


@functools.cache
def _get_deg_kernel():
    mesh = plsc.VectorSubcoreMesh(core_axis_name="c", subcore_axis_name="s")
    return functools.partial(
        pl.kernel,
        out_type=jax.ShapeDtypeStruct((NCORES, NN, CW), jnp.float32),
        mesh=mesh,
        scratch_types=[
            pltpu.VMEM((DEG_RPT, DBATCH), jnp.int32),
            pltpu.VMEM((DBATCH, CW), jnp.float32),
            pltpu.VMEM_SHARED((N_ACC, CW), jnp.float32),
        ],
    )(_deg_body)


def _deg_body(ones_hbm, zeros_hbm, dst_hbm, out_hbm, idx_v, ones_v, acc_sh):
    cid = lax.axis_index("c")
    sid = lax.axis_index("s")
    zoff = pl.multiple_of(sid * ACC_PT, 8)
    pltpu.sync_copy(zeros_hbm.at[pl.ds(zoff, ACC_PT)],
                    acc_sh.at[pl.ds(zoff, ACC_PT)])
    pltpu.sync_copy(ones_hbm, ones_v)
    base = pl.multiple_of(cid * (NTILES * DEG_RPT) + sid * DEG_RPT, 8)
    pltpu.sync_copy(dst_hbm.at[pl.ds(base, DEG_RPT)], idx_v)
    plsc.subcore_barrier()

    def body(j, carry):
        pltpu.sync_copy(ones_v, acc_sh.at[idx_v.at[j]], add=True)
        return carry

    lax.fori_loop(0, DEG_RPT, body, 0)
    plsc.subcore_barrier()
    _split_rows(sid, lambda off, cnt: pltpu.sync_copy(
        acc_sh.at[pl.ds(off, cnt)], out_hbm.at[cid].at[pl.ds(off, cnt)]))


@functools.cache
def _get_agg_kernel():
    mesh = plsc.VectorSubcoreMesh(core_axis_name="c", subcore_axis_name="s")
    return functools.partial(
        pl.kernel,
        out_type=jax.ShapeDtypeStruct((NCH, NN, CW), jnp.float32),
        mesh=mesh,
        scratch_types=[
            pltpu.VMEM((RPT // 2, BATCH), jnp.int32),
            pltpu.VMEM((RPT // 2, BATCH), jnp.int32),
            pltpu.VMEM((BATCH, CW), jnp.float32),
            pltpu.VMEM((BATCH, CW), jnp.float32),
            pltpu.VMEM_SHARED((N_ACC, CW), jnp.float32),
            pltpu.SemaphoreType.DMA,
            pltpu.SemaphoreType.DMA,
        ],
    )(_agg_body)


def _agg_body(hw_hbm, src_hbm, dst_hbm, out_hbm,
              src_v, dst_v, buf_a, buf_b, acc_sh, sem_a, sem_b):
    cid = lax.axis_index("c")
    sid = lax.axis_index("s")
    hrpt = RPT // 2

    def run_chunk(c):
        hw_c = hw_hbm.at[c]
        _split_rows(sid, lambda off, cnt: pltpu.sync_copy(
            hw_c.at[pl.ds(off, cnt)], acc_sh.at[pl.ds(off, cnt)]))
        plsc.subcore_barrier()

        def run_half(h):
            ioff = pl.multiple_of(sid * RPT + h * hrpt, 8)
            pltpu.sync_copy(src_hbm.at[pl.ds(ioff, hrpt)], src_v)
            pltpu.sync_copy(dst_hbm.at[pl.ds(ioff, hrpt)], dst_v)
            pltpu.async_copy(hw_c.at[src_v.at[0]], buf_a, sem_a)

            def step(j, buf, sem, nbuf, nsem):
                pltpu.make_async_copy(hw_c.at[src_v.at[0]], buf, sem).wait()

                @pl.when(j + 1 < hrpt)
                def _():
                    pltpu.async_copy(hw_c.at[src_v.at[j + 1]], nbuf, nsem)

                pltpu.sync_copy(buf, acc_sh.at[dst_v.at[j]], add=True)

            def pair(jj, carry):
                step(2 * jj, buf_a, sem_a, buf_b, sem_b)
                step(2 * jj + 1, buf_b, sem_b, buf_a, sem_a)
                return carry

            lax.fori_loop(0, hrpt // 2, pair, 0)

        run_half(0)
        run_half(1)
        plsc.subcore_barrier()
        _split_rows(sid, lambda off, cnt: pltpu.sync_copy(
            acc_sh.at[pl.ds(off, cnt)], out_hbm.at[c].at[pl.ds(off, cnt)]))
        plsc.subcore_barrier()

    for cc in range(NCH // 2):
        @pl.when(cid == 0)
        def _(cc=cc):
            run_chunk(cc)

        @pl.when(cid == 1)
        def _(cc=cc):
            run_chunk(NCH // 2 + cc)


_RB = 1000
_GRID = NN // _RB


def _dinv_of(degp_ref):
    deg = degp_ref[0, :, :1] + degp_ref[1, :, :1] + 1.0
    return 1.0 / jnp.sqrt(deg)


def _l1_body(x_ref, degp_ref, w_ref, out_ref):
    dinv = _dinv_of(degp_ref)
    hw = jnp.dot(x_ref[...], w_ref[...], preferred_element_type=jnp.float32,
                 precision=lax.Precision.HIGHEST)
    hwp = hw * dinv
    for c in range(NCH):
        out_ref[c] = hwp[:, c * CW:(c + 1) * CW]


def _layer_body(agg_ref, degp_ref, w_ref, b_ref, out_ref):
    dinv = _dinv_of(degp_ref)
    aggf = jnp.concatenate([agg_ref[c] for c in range(NCH)], axis=1)
    h = jnp.maximum(aggf * dinv + b_ref[...], 0.0)
    hw = jnp.dot(h, w_ref[...], preferred_element_type=jnp.float32,
                 precision=lax.Precision.HIGHEST)
    hwp = hw * dinv
    for c in range(NCH):
        out_ref[c] = hwp[:, c * CW:(c + 1) * CW]


def _h3_body(agg_ref, degp_ref, b_ref, out_ref):
    dinv = _dinv_of(degp_ref)
    aggf = jnp.concatenate([agg_ref[c] for c in range(NCH)], axis=1)
    out_ref[...] = jnp.maximum(aggf * dinv + b_ref[...], 0.0)


def _readout_body(h_ref, gw_ref, gb_ref, fw_ref, fb_ref, out_ref,
                  m_sc, s_sc, r_sc):
    i = pl.program_id(0)

    @pl.when(i == 0)
    def _():
        m_sc[0, 0] = -jnp.inf
        s_sc[0, 0] = 0.0
        r_sc[...] = jnp.zeros_like(r_sc)

    h = h_ref[...]
    g = jnp.dot(h, gw_ref[...], preferred_element_type=jnp.float32,
                precision=lax.Precision.HIGHEST) + gb_ref[0, 0]
    m_old = m_sc[0, 0]
    m_new = jnp.maximum(m_old, jnp.max(g))
    scale = jnp.exp(m_old - m_new)
    e = jnp.exp(g - m_new)
    s_new = s_sc[0, 0] * scale + jnp.sum(e)
    r_new = r_sc[...] * scale + jnp.sum(e * h, axis=0, keepdims=True)
    m_sc[0, 0] = m_new
    s_sc[0, 0] = s_new
    r_sc[...] = r_new

    @pl.when(i == _GRID - 1)
    def _():
        out_ref[...] = jnp.dot(r_new / s_new, fw_ref[...],
                               preferred_element_type=jnp.float32,
                               precision=lax.Precision.HIGHEST) + fb_ref[0, 0]


_degp_spec = pl.BlockSpec((2, _RB, CW), lambda i: (0, i, 0))
_chunks_spec = pl.BlockSpec((NCH, _RB, CW), lambda i: (0, i, 0))
_chunks_shape = jax.ShapeDtypeStruct((NCH, NN, CW), jnp.float32)


def _l1_call(x, degp, W1):
    return pl.pallas_call(
        _l1_body,
        grid=(_GRID,),
        in_specs=[
            pl.BlockSpec((_RB, DIN), lambda i: (i, 0)),
            _degp_spec,
            pl.BlockSpec((DIN, DH), lambda i: (0, 0)),
        ],
        out_specs=_chunks_spec,
        out_shape=_chunks_shape,
    )(x, degp, W1)


def _layer_call(agg, degp, W, b):
    return pl.pallas_call(
        _layer_body,
        grid=(_GRID,),
        in_specs=[
            _chunks_spec,
            _degp_spec,
            pl.BlockSpec((DH, DH), lambda i: (0, 0)),
            pl.BlockSpec((1, DH), lambda i: (0, 0)),
        ],
        out_specs=_chunks_spec,
        out_shape=_chunks_shape,
    )(agg, degp, W, b.reshape(1, DH))


def _h3_call(agg, degp, b):
    return pl.pallas_call(
        _h3_body,
        grid=(_GRID,),
        in_specs=[
            _chunks_spec,
            _degp_spec,
            pl.BlockSpec((1, DH), lambda i: (0, 0)),
        ],
        out_specs=pl.BlockSpec((_RB, DH), lambda i: (i, 0)),
        out_shape=jax.ShapeDtypeStruct((NN, DH), jnp.float32),
    )(agg, degp, b.reshape(1, DH))


def _readout_call(h3, gate_W, gate_b, fin_W, fin_b):
    return pl.pallas_call(
        _readout_body,
        grid=(_GRID,),
        in_specs=[
            pl.BlockSpec((_RB, DH), lambda i: (i, 0)),
            pl.BlockSpec((DH, 1), lambda i: (0, 0)),
            pl.BlockSpec((1, 1), lambda i: (0, 0)),
            pl.BlockSpec((DH, 1), lambda i: (0, 0)),
            pl.BlockSpec((1, 1), lambda i: (0, 0)),
        ],
        out_specs=pl.BlockSpec((1, 1), lambda i: (0, 0)),
        out_shape=jax.ShapeDtypeStruct((1, 1), jnp.float32),
        scratch_shapes=[
            pltpu.SMEM((1, 1), jnp.float32),
            pltpu.SMEM((1, 1), jnp.float32),
            pltpu.VMEM((1, DH), jnp.float32),
        ],
    )(h3, gate_W, gate_b.reshape(1, 1), fin_W, fin_b.reshape(1, 1))


def kernel(x, edge_index, W1, b1, W2, b2, W3, b3, gate_W, gate_b, fin_W, fin_b):
    order = jnp.argsort(edge_index[0])
    src = edge_index[0][order]
    dst = edge_index[1][order]
    pad = E_PAD - EE
    src2d = jnp.concatenate(
        [src, jnp.zeros((pad,), jnp.int32)]).reshape(E_PAD // BATCH, BATCH)
    dst_pad = jnp.concatenate([dst, jnp.full((pad,), NN, jnp.int32)])
    dst2d = dst_pad.reshape(E_PAD // BATCH, BATCH)
    dst2d_deg = dst_pad.reshape(DEG_ROWS, DBATCH)

    ones128 = jnp.ones((DBATCH, CW), jnp.float32)
    zeros128 = jnp.zeros((N_ACC, CW), jnp.float32)
    degp = _get_deg_kernel()(ones128, zeros128, dst2d_deg)

    hw1 = _l1_call(x, degp, W1)
    agg1 = _get_agg_kernel()(hw1, src2d, dst2d)
    hw2 = _layer_call(agg1, degp, W2, b1)
    agg2 = _get_agg_kernel()(hw2, src2d, dst2d)
    hw3 = _layer_call(agg2, degp, W3, b2)
    agg3 = _get_agg_kernel()(hw3, src2d, dst2d)
    h3 = _h3_call(agg3, degp, b3)
    return _readout_call(h3, gate_W, gate_b, fin_W, fin_b)


# speedup vs baseline: 1.3313x; 1.3313x over previous
import functools

import jax
import jax.numpy as jnp
from jax import lax
from jax.experimental import pallas as pl
from jax.experimental.pallas import tpu as pltpu
from jax.experimental.pallas import tpu_sc as plsc

NN = 10000
EE = 160000
DIN = 256
DH = 512

NTILES = 16
NCORES = 2
BATCH = 128
RPT = 80
E_PAD = NTILES * BATCH * RPT
N_ACC = 10112
ROWS_PT = 632
ROWS_LAST = NN - ROWS_PT * (NTILES - 1)
ACC_PT = N_ACC // NTILES
DBATCH = 128
DEG_ROWS = E_PAD // DBATCH
NCH = 4
CW = DH // NCH
DEG_RPT = DEG_ROWS // (NCORES * NTILES)


def _split_rows(sid, fn):
    @pl.when(sid < NTILES - 1)
    def _():
        fn(pl.multiple_of(sid * ROWS_PT, 8), ROWS_PT)

    @pl.when(sid == NTILES - 1)
    def _():
        fn((NTILES - 1) * ROWS_PT, ROWS_LAST)


@functools.cache
def _get_deg_kernel():
    mesh = plsc.VectorSubcoreMesh(core_axis_name="c", subcore_axis_name="s")
    return functools.partial(
        pl.kernel,
        out_type=jax.ShapeDtypeStruct((NCORES, NN, CW), jnp.float32),
        mesh=mesh,
        scratch_types=[
            pltpu.VMEM((DEG_RPT, DBATCH), jnp.int32),
            pltpu.VMEM((DBATCH, CW), jnp.float32),
            pltpu.VMEM_SHARED((N_ACC, CW), jnp.float32),
        ],
    )(_deg_body)


def _deg_body(ones_hbm, zeros_hbm, dst_hbm, out_hbm, idx_v, ones_v, acc_sh):
    cid = lax.axis_index("c")
    sid = lax.axis_index("s")
    zoff = pl.multiple_of(sid * ACC_PT, 8)
    pltpu.sync_copy(zeros_hbm.at[pl.ds(zoff, ACC_PT)],
                    acc_sh.at[pl.ds(zoff, ACC_PT)])
    pltpu.sync_copy(ones_hbm, ones_v)
    base = pl.multiple_of(cid * (NTILES * DEG_RPT) + sid * DEG_RPT, 8)
    pltpu.sync_copy(dst_hbm.at[pl.ds(base, DEG_RPT)], idx_v)
    plsc.subcore_barrier()

    def body(j, carry):
        pltpu.sync_copy(ones_v, acc_sh.at[idx_v.at[j]], add=True)
        return carry

    lax.fori_loop(0, DEG_RPT, body, 0)
    plsc.subcore_barrier()
    _split_rows(sid, lambda off, cnt: pltpu.sync_copy(
        acc_sh.at[pl.ds(off, cnt)], out_hbm.at[cid].at[pl.ds(off, cnt)]))


@functools.cache
def _get_agg_kernel():
    mesh = plsc.VectorSubcoreMesh(core_axis_name="c", subcore_axis_name="s")
    return functools.partial(
        pl.kernel,
        out_type=jax.ShapeDtypeStruct((NCH, NN, CW), jnp.float32),
        mesh=mesh,
        scratch_types=[
            pltpu.VMEM((RPT // 2, BATCH), jnp.int32),
            pltpu.VMEM((RPT // 2, BATCH), jnp.int32),
            pltpu.VMEM((BATCH, CW), jnp.float32),
            pltpu.VMEM((BATCH, CW), jnp.float32),
            pltpu.VMEM_SHARED((N_ACC, CW), jnp.float32),
            pltpu.SemaphoreType.DMA,
            pltpu.SemaphoreType.DMA,
        ],
    )(_agg_body)


def _agg_body(hw_hbm, src_hbm, dst_hbm, out_hbm,
              src_v, dst_v, buf_a, buf_b, acc_sh, sem_a, sem_b):
    cid = lax.axis_index("c")
    sid = lax.axis_index("s")
    hrpt = RPT // 2

    def run_chunk(c):
        hw_c = hw_hbm.at[c]
        _split_rows(sid, lambda off, cnt: pltpu.sync_copy(
            hw_c.at[pl.ds(off, cnt)], acc_sh.at[pl.ds(off, cnt)]))
        plsc.subcore_barrier()

        def run_half(h):
            ioff = pl.multiple_of(sid * RPT + h * hrpt, 8)
            pltpu.sync_copy(src_hbm.at[pl.ds(ioff, hrpt)], src_v)
            pltpu.sync_copy(dst_hbm.at[pl.ds(ioff, hrpt)], dst_v)
            pltpu.async_copy(hw_c.at[src_v.at[0]], buf_a, sem_a)

            def step(j, buf, sem, nbuf, nsem):
                pltpu.make_async_copy(hw_c.at[src_v.at[0]], buf, sem).wait()

                @pl.when(j + 1 < hrpt)
                def _():
                    pltpu.async_copy(hw_c.at[src_v.at[j + 1]], nbuf, nsem)

                pltpu.sync_copy(buf, acc_sh.at[dst_v.at[j]], add=True)

            def pair(jj, carry):
                step(2 * jj, buf_a, sem_a, buf_b, sem_b)
                step(2 * jj + 1, buf_b, sem_b, buf_a, sem_a)
                return carry

            lax.fori_loop(0, hrpt // 2, pair, 0)

        run_half(0)
        run_half(1)
        plsc.subcore_barrier()
        _split_rows(sid, lambda off, cnt: pltpu.sync_copy(
            acc_sh.at[pl.ds(off, cnt)], out_hbm.at[c].at[pl.ds(off, cnt)]))
        plsc.subcore_barrier()

    for cc in range(NCH // 2):
        @pl.when(cid == 0)
        def _(cc=cc):
            run_chunk(cc)

        @pl.when(cid == 1)
        def _(cc=cc):
            run_chunk(NCH // 2 + cc)


_RB = 1000
_GRID = NN // _RB


def _dinv_of(degp_ref):
    deg = degp_ref[0, :, :1] + degp_ref[1, :, :1] + 1.0
    return 1.0 / jnp.sqrt(deg)


def _l1_body(x_ref, degp_ref, w_ref, out_ref):
    dinv = _dinv_of(degp_ref)
    hw = jnp.dot(x_ref[...], w_ref[...], preferred_element_type=jnp.float32)
    hwp = hw * dinv
    for c in range(NCH):
        out_ref[c] = hwp[:, c * CW:(c + 1) * CW]


def _layer_body(agg_ref, degp_ref, w_ref, b_ref, out_ref):
    dinv = _dinv_of(degp_ref)
    aggf = jnp.concatenate([agg_ref[c] for c in range(NCH)], axis=1)
    h = jnp.maximum(aggf * dinv + b_ref[...], 0.0)
    hw = jnp.dot(h, w_ref[...], preferred_element_type=jnp.float32)
    hwp = hw * dinv
    for c in range(NCH):
        out_ref[c] = hwp[:, c * CW:(c + 1) * CW]


def _h3_body(agg_ref, degp_ref, b_ref, out_ref):
    dinv = _dinv_of(degp_ref)
    aggf = jnp.concatenate([agg_ref[c] for c in range(NCH)], axis=1)
    out_ref[...] = jnp.maximum(aggf * dinv + b_ref[...], 0.0)


def _readout_body(h_ref, gw_ref, gb_ref, fw_ref, fb_ref, out_ref,
                  m_sc, s_sc, r_sc):
    i = pl.program_id(0)

    @pl.when(i == 0)
    def _():
        m_sc[0, 0] = -jnp.inf
        s_sc[0, 0] = 0.0
        r_sc[...] = jnp.zeros_like(r_sc)

    h = h_ref[...]
    g = jnp.dot(h, gw_ref[...], preferred_element_type=jnp.float32) + gb_ref[0, 0]
    m_old = m_sc[0, 0]
    m_new = jnp.maximum(m_old, jnp.max(g))
    scale = jnp.exp(m_old - m_new)
    e = jnp.exp(g - m_new)
    s_new = s_sc[0, 0] * scale + jnp.sum(e)
    r_new = r_sc[...] * scale + jnp.sum(e * h, axis=0, keepdims=True)
    m_sc[0, 0] = m_new
    s_sc[0, 0] = s_new
    r_sc[...] = r_new

    @pl.when(i == _GRID - 1)
    def _():
        out_ref[...] = jnp.dot(r_new / s_new, fw_ref[...],
                               preferred_element_type=jnp.float32) + fb_ref[0, 0]


_degp_spec = pl.BlockSpec((2, _RB, CW), lambda i: (0, i, 0))
_chunks_spec = pl.BlockSpec((NCH, _RB, CW), lambda i: (0, i, 0))
_chunks_shape = jax.ShapeDtypeStruct((NCH, NN, CW), jnp.float32)


def _l1_call(x, degp, W1):
    return pl.pallas_call(
        _l1_body,
        grid=(_GRID,),
        in_specs=[
            pl.BlockSpec((_RB, DIN), lambda i: (i, 0)),
            _degp_spec,
            pl.BlockSpec((DIN, DH), lambda i: (0, 0)),
        ],
        out_specs=_chunks_spec,
        out_shape=_chunks_shape,
    )(x, degp, W1)


def _layer_call(agg, degp, W, b):
    return pl.pallas_call(
        _layer_body,
        grid=(_GRID,),
        in_specs=[
            _chunks_spec,
            _degp_spec,
            pl.BlockSpec((DH, DH), lambda i: (0, 0)),
            pl.BlockSpec((1, DH), lambda i: (0, 0)),
        ],
        out_specs=_chunks_spec,
        out_shape=_chunks_shape,
    )(agg, degp, W, b.reshape(1, DH))


def _h3_call(agg, degp, b):
    return pl.pallas_call(
        _h3_body,
        grid=(_GRID,),
        in_specs=[
            _chunks_spec,
            _degp_spec,
            pl.BlockSpec((1, DH), lambda i: (0, 0)),
        ],
        out_specs=pl.BlockSpec((_RB, DH), lambda i: (i, 0)),
        out_shape=jax.ShapeDtypeStruct((NN, DH), jnp.float32),
    )(agg, degp, b.reshape(1, DH))


def _readout_call(h3, gate_W, gate_b, fin_W, fin_b):
    return pl.pallas_call(
        _readout_body,
        grid=(_GRID,),
        in_specs=[
            pl.BlockSpec((_RB, DH), lambda i: (i, 0)),
            pl.BlockSpec((DH, 1), lambda i: (0, 0)),
            pl.BlockSpec((1, 1), lambda i: (0, 0)),
            pl.BlockSpec((DH, 1), lambda i: (0, 0)),
            pl.BlockSpec((1, 1), lambda i: (0, 0)),
        ],
        out_specs=pl.BlockSpec((1, 1), lambda i: (0, 0)),
        out_shape=jax.ShapeDtypeStruct((1, 1), jnp.float32),
        scratch_shapes=[
            pltpu.SMEM((1, 1), jnp.float32),
            pltpu.SMEM((1, 1), jnp.float32),
            pltpu.VMEM((1, DH), jnp.float32),
        ],
    )(h3, gate_W, gate_b.reshape(1, 1), fin_W, fin_b.reshape(1, 1))


def kernel(x, edge_index, W1, b1, W2, b2, W3, b3, gate_W, gate_b, fin_W, fin_b):
    src = edge_index[0]
    dst = edge_index[1]
    pad = E_PAD - EE
    src2d = jnp.concatenate(
        [src, jnp.zeros((pad,), jnp.int32)]).reshape(E_PAD // BATCH, BATCH)
    dst_pad = jnp.concatenate([dst, jnp.full((pad,), NN, jnp.int32)])
    dst2d = dst_pad.reshape(E_PAD // BATCH, BATCH)
    dst2d_deg = dst_pad.reshape(DEG_ROWS, DBATCH)

    ones128 = jnp.ones((DBATCH, CW), jnp.float32)
    zeros128 = jnp.zeros((N_ACC, CW), jnp.float32)
    degp = _get_deg_kernel()(ones128, zeros128, dst2d_deg)

    hw1 = _l1_call(x, degp, W1)
    agg1 = _get_agg_kernel()(hw1, src2d, dst2d)
    hw2 = _layer_call(agg1, degp, W2, b1)
    agg2 = _get_agg_kernel()(hw2, src2d, dst2d)
    hw3 = _layer_call(agg2, degp, W3, b2)
    agg3 = _get_agg_kernel()(hw3, src2d, dst2d)
    h3 = _h3_call(agg3, degp, b3)
    return _readout_call(h3, gate_W, gate_b, fin_W, fin_b)


# speedup vs baseline: 1.3317x; 1.0003x over previous
import functools

import jax
import jax.numpy as jnp
from jax import lax
from jax.experimental import pallas as pl
from jax.experimental.pallas import tpu as pltpu
from jax.experimental.pallas import tpu_sc as plsc

NN = 10000
EE = 160000
DIN = 256
DH = 512

NTILES = 16
NCORES = 2
BATCH = 128
RPT = 80
E_PAD = NTILES * BATCH * RPT
N_ACC = 10112
ROWS_PT = 632
ROWS_LAST = NN - ROWS_PT * (NTILES - 1)
ACC_PT = N_ACC // NTILES
DBATCH = 128
DEG_ROWS = E_PAD // DBATCH
NCH = 4
CW = DH // NCH
DEG_RPT = DEG_ROWS // (NCORES * NTILES)


def _split_rows(sid, fn):
    @pl.when(sid < NTILES - 1)
    def _():
        fn(pl.multiple_of(sid * ROWS_PT, 8), ROWS_PT)

    @pl.when(sid == NTILES - 1)
    def _():
        fn((NTILES - 1) * ROWS_PT, ROWS_LAST)


@functools.cache
def _get_deg_kernel():
    mesh = plsc.VectorSubcoreMesh(core_axis_name="c", subcore_axis_name="s")
    return functools.partial(
        pl.kernel,
        out_type=jax.ShapeDtypeStruct((NCORES, NN, CW), jnp.float32),
        mesh=mesh,
        scratch_types=[
            pltpu.VMEM((DEG_RPT, DBATCH), jnp.int32),
            pltpu.VMEM((DBATCH, CW), jnp.float32),
            pltpu.VMEM_SHARED((N_ACC, CW), jnp.float32),
        ],
    )(_deg_body)


def _deg_body(ones_hbm, zeros_hbm, dst_hbm, out_hbm, idx_v, ones_v, acc_sh):
    cid = lax.axis_index("c")
    sid = lax.axis_index("s")
    zoff = pl.multiple_of(sid * ACC_PT, 8)
    pltpu.sync_copy(zeros_hbm.at[pl.ds(zoff, ACC_PT)],
                    acc_sh.at[pl.ds(zoff, ACC_PT)])
    pltpu.sync_copy(ones_hbm, ones_v)
    base = pl.multiple_of(cid * (NTILES * DEG_RPT) + sid * DEG_RPT, 8)
    pltpu.sync_copy(dst_hbm.at[pl.ds(base, DEG_RPT)], idx_v)
    plsc.subcore_barrier()

    def body(j, carry):
        pltpu.sync_copy(ones_v, acc_sh.at[idx_v.at[j]], add=True)
        return carry

    lax.fori_loop(0, DEG_RPT, body, 0)
    plsc.subcore_barrier()
    _split_rows(sid, lambda off, cnt: pltpu.sync_copy(
        acc_sh.at[pl.ds(off, cnt)], out_hbm.at[cid].at[pl.ds(off, cnt)]))


@functools.cache
def _get_agg_kernel():
    mesh = plsc.VectorSubcoreMesh(core_axis_name="c", subcore_axis_name="s")
    return functools.partial(
        pl.kernel,
        out_type=jax.ShapeDtypeStruct((NCH, NN, CW), jnp.float32),
        mesh=mesh,
        scratch_types=[
            pltpu.VMEM((RPT // 2, BATCH), jnp.int32),
            pltpu.VMEM((RPT // 2, BATCH), jnp.int32),
            pltpu.VMEM((BATCH, CW), jnp.float32),
            pltpu.VMEM((BATCH, CW), jnp.float32),
            pltpu.VMEM_SHARED((N_ACC, CW), jnp.float32),
            pltpu.SemaphoreType.DMA,
            pltpu.SemaphoreType.DMA,
            pltpu.SemaphoreType.DMA,
            pltpu.SemaphoreType.DMA,
        ],
    )(_agg_body)


def _agg_body(hw_hbm, src_hbm, dst_hbm, out_hbm,
              src_v, dst_v, buf_a, buf_b, acc_sh,
              sem_a, sem_b, sem_a2, sem_b2):
    cid = lax.axis_index("c")
    sid = lax.axis_index("s")
    hrpt = RPT // 2

    def run_chunk(c):
        hw_c = hw_hbm.at[c]
        _split_rows(sid, lambda off, cnt: pltpu.sync_copy(
            hw_c.at[pl.ds(off, cnt)], acc_sh.at[pl.ds(off, cnt)]))
        plsc.subcore_barrier()

        def run_half(h):
            ioff = pl.multiple_of(sid * RPT + h * hrpt, 8)
            pltpu.sync_copy(src_hbm.at[pl.ds(ioff, hrpt)], src_v)
            pltpu.sync_copy(dst_hbm.at[pl.ds(ioff, hrpt)], dst_v)
            def issue(j, buf, sem, sem2):
                pltpu.async_copy(hw_c.at[src_v.at[j, pl.ds(0, 64)]],
                                 buf.at[pl.ds(0, 64)], sem)
                pltpu.async_copy(hw_c.at[src_v.at[j, pl.ds(64, 64)]],
                                 buf.at[pl.ds(64, 64)], sem2)

            def wait(buf, sem, sem2):
                pltpu.make_async_copy(hw_c.at[src_v.at[0, pl.ds(0, 64)]],
                                      buf.at[pl.ds(0, 64)], sem).wait()
                pltpu.make_async_copy(hw_c.at[src_v.at[0, pl.ds(64, 64)]],
                                      buf.at[pl.ds(64, 64)], sem2).wait()

            issue(0, buf_a, sem_a, sem_a2)

            def step(j, buf, sem, sem2, nbuf, nsem, nsem2):
                wait(buf, sem, sem2)

                @pl.when(j + 1 < hrpt)
                def _():
                    issue(j + 1, nbuf, nsem, nsem2)

                pltpu.sync_copy(buf, acc_sh.at[dst_v.at[j]], add=True)

            def pair(jj, carry):
                step(2 * jj, buf_a, sem_a, sem_a2, buf_b, sem_b, sem_b2)
                step(2 * jj + 1, buf_b, sem_b, sem_b2, buf_a, sem_a, sem_a2)
                return carry

            lax.fori_loop(0, hrpt // 2, pair, 0)

        run_half(0)
        run_half(1)
        plsc.subcore_barrier()
        _split_rows(sid, lambda off, cnt: pltpu.sync_copy(
            acc_sh.at[pl.ds(off, cnt)], out_hbm.at[c].at[pl.ds(off, cnt)]))
        plsc.subcore_barrier()

    for cc in range(NCH // 2):
        @pl.when(cid == 0)
        def _(cc=cc):
            run_chunk(cc)

        @pl.when(cid == 1)
        def _(cc=cc):
            run_chunk(NCH // 2 + cc)


_RB = 1000
_GRID = NN // _RB


def _dinv_of(degp_ref):
    deg = degp_ref[0, :, :1] + degp_ref[1, :, :1] + 1.0
    return 1.0 / jnp.sqrt(deg)


def _l1_body(x_ref, degp_ref, w_ref, out_ref):
    dinv = _dinv_of(degp_ref)
    hw = jnp.dot(x_ref[...], w_ref[...], preferred_element_type=jnp.float32)
    hwp = hw * dinv
    for c in range(NCH):
        out_ref[c] = hwp[:, c * CW:(c + 1) * CW]


def _layer_body(agg_ref, degp_ref, w_ref, b_ref, out_ref):
    dinv = _dinv_of(degp_ref)
    aggf = jnp.concatenate([agg_ref[c] for c in range(NCH)], axis=1)
    h = jnp.maximum(aggf * dinv + b_ref[...], 0.0)
    hw = jnp.dot(h, w_ref[...], preferred_element_type=jnp.float32)
    hwp = hw * dinv
    for c in range(NCH):
        out_ref[c] = hwp[:, c * CW:(c + 1) * CW]


def _h3_body(agg_ref, degp_ref, b_ref, out_ref):
    dinv = _dinv_of(degp_ref)
    aggf = jnp.concatenate([agg_ref[c] for c in range(NCH)], axis=1)
    out_ref[...] = jnp.maximum(aggf * dinv + b_ref[...], 0.0)


def _readout_body(h_ref, gw_ref, gb_ref, fw_ref, fb_ref, out_ref,
                  m_sc, s_sc, r_sc):
    i = pl.program_id(0)

    @pl.when(i == 0)
    def _():
        m_sc[0, 0] = -jnp.inf
        s_sc[0, 0] = 0.0
        r_sc[...] = jnp.zeros_like(r_sc)

    h = h_ref[...]
    g = jnp.dot(h, gw_ref[...], preferred_element_type=jnp.float32) + gb_ref[0, 0]
    m_old = m_sc[0, 0]
    m_new = jnp.maximum(m_old, jnp.max(g))
    scale = jnp.exp(m_old - m_new)
    e = jnp.exp(g - m_new)
    s_new = s_sc[0, 0] * scale + jnp.sum(e)
    r_new = r_sc[...] * scale + jnp.sum(e * h, axis=0, keepdims=True)
    m_sc[0, 0] = m_new
    s_sc[0, 0] = s_new
    r_sc[...] = r_new

    @pl.when(i == _GRID - 1)
    def _():
        out_ref[...] = jnp.dot(r_new / s_new, fw_ref[...],
                               preferred_element_type=jnp.float32) + fb_ref[0, 0]


_degp_spec = pl.BlockSpec((2, _RB, CW), lambda i: (0, i, 0))
_chunks_spec = pl.BlockSpec((NCH, _RB, CW), lambda i: (0, i, 0))
_chunks_shape = jax.ShapeDtypeStruct((NCH, NN, CW), jnp.float32)


def _l1_call(x, degp, W1):
    return pl.pallas_call(
        _l1_body,
        grid=(_GRID,),
        in_specs=[
            pl.BlockSpec((_RB, DIN), lambda i: (i, 0)),
            _degp_spec,
            pl.BlockSpec((DIN, DH), lambda i: (0, 0)),
        ],
        out_specs=_chunks_spec,
        out_shape=_chunks_shape,
    )(x, degp, W1)


def _layer_call(agg, degp, W, b):
    return pl.pallas_call(
        _layer_body,
        grid=(_GRID,),
        in_specs=[
            _chunks_spec,
            _degp_spec,
            pl.BlockSpec((DH, DH), lambda i: (0, 0)),
            pl.BlockSpec((1, DH), lambda i: (0, 0)),
        ],
        out_specs=_chunks_spec,
        out_shape=_chunks_shape,
    )(agg, degp, W, b.reshape(1, DH))


def _h3_call(agg, degp, b):
    return pl.pallas_call(
        _h3_body,
        grid=(_GRID,),
        in_specs=[
            _chunks_spec,
            _degp_spec,
            pl.BlockSpec((1, DH), lambda i: (0, 0)),
        ],
        out_specs=pl.BlockSpec((_RB, DH), lambda i: (i, 0)),
        out_shape=jax.ShapeDtypeStruct((NN, DH), jnp.float32),
    )(agg, degp, b.reshape(1, DH))


def _readout_call(h3, gate_W, gate_b, fin_W, fin_b):
    return pl.pallas_call(
        _readout_body,
        grid=(_GRID,),
        in_specs=[
            pl.BlockSpec((_RB, DH), lambda i: (i, 0)),
            pl.BlockSpec((DH, 1), lambda i: (0, 0)),
            pl.BlockSpec((1, 1), lambda i: (0, 0)),
            pl.BlockSpec((DH, 1), lambda i: (0, 0)),
            pl.BlockSpec((1, 1), lambda i: (0, 0)),
        ],
        out_specs=pl.BlockSpec((1, 1), lambda i: (0, 0)),
        out_shape=jax.ShapeDtypeStruct((1, 1), jnp.float32),
        scratch_shapes=[
            pltpu.SMEM((1, 1), jnp.float32),
            pltpu.SMEM((1, 1), jnp.float32),
            pltpu.VMEM((1, DH), jnp.float32),
        ],
    )(h3, gate_W, gate_b.reshape(1, 1), fin_W, fin_b.reshape(1, 1))


def kernel(x, edge_index, W1, b1, W2, b2, W3, b3, gate_W, gate_b, fin_W, fin_b):
    src = edge_index[0]
    dst = edge_index[1]
    pad = E_PAD - EE
    src2d = jnp.concatenate(
        [src, jnp.zeros((pad,), jnp.int32)]).reshape(E_PAD // BATCH, BATCH)
    dst_pad = jnp.concatenate([dst, jnp.full((pad,), NN, jnp.int32)])
    dst2d = dst_pad.reshape(E_PAD // BATCH, BATCH)
    dst2d_deg = dst_pad.reshape(DEG_ROWS, DBATCH)

    ones128 = jnp.ones((DBATCH, CW), jnp.float32)
    zeros128 = jnp.zeros((N_ACC, CW), jnp.float32)
    degp = _get_deg_kernel()(ones128, zeros128, dst2d_deg)

    hw1 = _l1_call(x, degp, W1)
    agg1 = _get_agg_kernel()(hw1, src2d, dst2d)
    hw2 = _layer_call(agg1, degp, W2, b1)
    agg2 = _get_agg_kernel()(hw2, src2d, dst2d)
    hw3 = _layer_call(agg2, degp, W3, b2)
    agg3 = _get_agg_kernel()(hw3, src2d, dst2d)
    h3 = _h3_call(agg3, degp, b3)
    return _readout_call(h3, gate_W, gate_b, fin_W, fin_b)
